# baseline (speedup 1.0000x reference)
import jax
import jax.numpy as jnp
from jax.experimental import pallas as pl
from jax.experimental.pallas import tpu as pltpu

_TI = 400
_TS = 80


def _body(x_ref, adj_ref, adjs_ref, w1_ref, w2_ref, wo_ref, bo_ref, out_ref,
          s1, c_acc, h1_acc):
    i = pl.program_id(0)
    nsteps = pl.num_programs(0)
    n = s1.shape[0]
    nbig = (n - 5 * _TS) // _TI
    tx = x_ref.shape[0]
    nxblk = n // tx

    @pl.when(i < nxblk)
    def _prologue():
        s1[pl.ds(i * tx, tx), :] = jnp.dot(
            x_ref[...], w1_ref[...], preferred_element_type=jnp.float32)

    @pl.when(jnp.logical_and(i >= nxblk, i < nxblk + nbig))
    def _stream_big():
        r = i - nxblk
        blk = adj_ref[...]
        h1 = jnp.maximum(
            jnp.dot(blk, s1[...], preferred_element_type=jnp.float32), 0.0)
        h1_acc[pl.ds(r * _TI, _TI), :] = h1
        colsum = jnp.sum(blk, axis=0, keepdims=True)
        c_acc[...] = jnp.where(r == 0, colsum, c_acc[...] + colsum)

    @pl.when(i >= nxblk + nbig)
    def _stream_tail():
        rr = i - (nxblk + nbig)
        blk = adjs_ref[...]
        h1 = jnp.maximum(
            jnp.dot(blk, s1[...], preferred_element_type=jnp.float32), 0.0)
        h1_acc[pl.ds(nbig * _TI + rr * _TS, _TS), :] = h1
        c_acc[...] = c_acc[...] + jnp.sum(blk, axis=0, keepdims=True)

    @pl.when(i == nsteps - 1)
    def _epilogue():
        pooled = jnp.dot(c_acc[...], h1_acc[...],
                         preferred_element_type=jnp.float32)
        z = jnp.dot(pooled, w2_ref[...],
                    preferred_element_type=jnp.float32)
        o = jnp.dot(z, wo_ref[...],
                    preferred_element_type=jnp.float32) + bo_ref[...]
        out_ref[...] = jax.nn.sigmoid(o)


def kernel(x, adj, W1, W2, Wo, bo):
    n, d_in = x.shape
    h1_dim = W1.shape[1]
    h2_dim = W2.shape[1]

    tx = 2000
    nxblk = n // tx
    nbig = (n - 5 * _TS) // _TI
    nsteps = nxblk + nbig + 5
    tail0 = (nbig * _TI) // _TS

    out = pl.pallas_call(
        _body,
        grid=(nsteps,),
        in_specs=[
            pl.BlockSpec((tx, d_in), lambda i: (jnp.minimum(i, nxblk - 1), 0)),
            pl.BlockSpec((_TI, n),
                         lambda i: (jnp.clip(i - nxblk, 0, nbig - 1), 0)),
            pl.BlockSpec((_TS, n),
                         lambda i: (jnp.maximum(i - (nxblk + nbig), 0) + tail0, 0)),
            pl.BlockSpec((d_in, h1_dim), lambda i: (0, 0)),
            pl.BlockSpec((h1_dim, h2_dim), lambda i: (0, 0)),
            pl.BlockSpec((h2_dim, 1), lambda i: (0, 0)),
            pl.BlockSpec((1, 1), lambda i: (0, 0)),
        ],
        out_specs=pl.BlockSpec((1, 1), lambda i: (0, 0)),
        out_shape=jax.ShapeDtypeStruct((1, 1), jnp.float32),
        scratch_shapes=[
            pltpu.VMEM((n, h1_dim), jnp.float32),
            pltpu.VMEM((1, n), jnp.float32),
            pltpu.VMEM((n, h1_dim), jnp.float32),
        ],
        compiler_params=pltpu.CompilerParams(
            dimension_semantics=("arbitrary",),
            vmem_limit_bytes=62914560),
    )(x, adj, adj, W1, W2, Wo, bo.reshape(1, 1))

    return out.reshape(1)
